# Initial kernel scaffold; baseline (speedup 1.0000x reference)
import jax
import jax.numpy as jnp
from jax.experimental import pallas as pl


def kernel(box_pred, confidence_pred):
    raise NotImplementedError("write your pallas kernel here")


# speedup vs baseline: 18.6325x; 18.6325x over previous
import jax
import jax.numpy as jnp
from jax.experimental import pallas as pl
from jax.experimental.pallas import tpu as pltpu

_IOU_THR = 0.5
_CONF_THR = 0.05
_MAX_DET = 100
_N_PAD = 1024


def _nms_body(conf_ref, box_ref, conf_o, cls_o, box_o, num_o,
              s_ref, cs_ref, ci_ref):
    B, C, NP = conf_ref.shape
    T = cs_ref.shape[0]
    N = 1000
    f32 = jnp.float32
    i32 = jnp.int32
    NEG = f32(-jnp.inf)

    conf_o[...] = jnp.zeros(conf_o.shape, f32)
    cls_o[...] = jnp.zeros(cls_o.shape, f32)
    box_o[...] = jnp.zeros(box_o.shape, f32)
    cs_ref[...] = jnp.full(cs_ref.shape, -1.0, f32)
    ci_ref[...] = jnp.zeros(ci_ref.shape, i32)

    n_io3 = jax.lax.broadcasted_iota(i32, (B, C, NP), 2)

    z = conf_ref[...]
    zmax = jnp.max(z, axis=1, keepdims=True)
    e = jnp.exp(z - zmax)
    se = jnp.sum(e, axis=1, keepdims=True)
    s = e / se
    s_ref[...] = jnp.where(n_io3 < N, s, NEG)

    y1 = box_ref[:, 0, :]
    x1 = box_ref[:, 1, :]
    y2 = box_ref[:, 2, :]
    x2 = box_ref[:, 3, :]
    area = jnp.maximum(y2 - y1, 0.0) * jnp.maximum(x2 - x1, 0.0)

    def nms_cond(c):
        t, go = c
        return go & (t < T)

    def nms_step(c):
        t, _ = c
        sw = s_ref[...]
        m = jnp.max(sw, axis=2)
        idx = jnp.min(jnp.where(sw == m[:, :, None], n_io3, NP), axis=2)
        valid = m > _CONF_THR
        oh = n_io3 == idx[:, :, None]
        sel_y1 = jnp.sum(jnp.where(oh, y1[:, None, :], 0.0), axis=2)
        sel_x1 = jnp.sum(jnp.where(oh, x1[:, None, :], 0.0), axis=2)
        sel_y2 = jnp.sum(jnp.where(oh, y2[:, None, :], 0.0), axis=2)
        sel_x2 = jnp.sum(jnp.where(oh, x2[:, None, :], 0.0), axis=2)
        a_sel = (jnp.maximum(sel_y2 - sel_y1, 0.0) *
                 jnp.maximum(sel_x2 - sel_x1, 0.0))
        iy1 = jnp.maximum(sel_y1[:, :, None], y1[:, None, :])
        ix1 = jnp.maximum(sel_x1[:, :, None], x1[:, None, :])
        iy2 = jnp.minimum(sel_y2[:, :, None], y2[:, None, :])
        ix2 = jnp.minimum(sel_x2[:, :, None], x2[:, None, :])
        inter = (jnp.maximum(iy2 - iy1, 0.0) *
                 jnp.maximum(ix2 - ix1, 0.0))
        union = a_sel[:, :, None] + area[:, None, :] - inter
        iou = inter / (union + 1e-8)
        supp = (iou > _IOU_THR) | oh
        s_ref[...] = jnp.where(valid[:, :, None] & supp, NEG, sw)
        cs_ref[t] = jnp.where(valid, m, -1.0)
        ci_ref[t] = idx
        return t + 1, jnp.any(valid)

    jax.lax.while_loop(nms_cond, nms_step,
                       (jnp.array(0, i32), jnp.array(True)))

    t_io = jax.lax.broadcasted_iota(i32, (T, B, C), 0)
    c_io3 = jax.lax.broadcasted_iota(i32, (T, B, C), 2)
    c_io2 = jax.lax.broadcasted_iota(i32, (B, C), 1)
    n_io2 = jax.lax.broadcasted_iota(i32, (B, NP), 1)

    def topk_cond(c):
        k, go = c
        return go & (k < T)

    def topk_step(c):
        k, _ = c
        cs = cs_ref[...]
        m_t = jnp.max(cs, axis=0)
        tstar = jnp.min(jnp.where(cs == m_t[None], t_io, T), axis=0)
        m_b = jnp.max(m_t, axis=1)
        cstar = jnp.min(jnp.where(m_t == m_b[:, None], c_io2, C), axis=1)
        tsel = jnp.min(jnp.where(c_io2 == cstar[:, None], tstar, 10000),
                       axis=1)
        oh3 = ((t_io == tsel[None, :, None]) &
               (c_io3 == cstar[None, :, None]))
        bidx = jnp.sum(jnp.sum(jnp.where(oh3, ci_ref[...], 0), axis=0),
                       axis=1)
        cs_ref[...] = jnp.where(oh3, NEG, cs)
        valid = m_b > 0.0
        conf_o[k] = jnp.where(valid, m_b, 0.0).reshape(1, B)
        cls_o[k] = jnp.where(valid, cstar.astype(f32), 0.0).reshape(1, B)
        ohn = (n_io2 == bidx[:, None]) & valid[:, None]
        rows = [
            jnp.sum(jnp.where(ohn, box_ref[:, j, :], 0.0),
                    axis=1).reshape(1, B)
            for j in range(4)
        ]
        box_o[k] = jnp.concatenate(rows, axis=0)
        return k + 1, jnp.any(valid)

    jax.lax.while_loop(topk_cond, topk_step,
                       (jnp.array(0, i32), jnp.array(True)))

    num_o[...] = jnp.sum((conf_o[...] > 0.0).astype(i32), axis=0)


def kernel(box_pred, confidence_pred):
    B, N, C = confidence_pred.shape
    NP = _N_PAD
    T = _MAX_DET
    conf_t = jnp.transpose(confidence_pred, (0, 2, 1))
    conf_t = jnp.pad(conf_t, ((0, 0), (0, 0), (0, NP - N)))
    box_t = jnp.transpose(box_pred, (0, 2, 1))
    box_t = jnp.pad(box_t, ((0, 0), (0, 0), (0, NP - N)))

    conf_o, cls_o, box_o, num_o = pl.pallas_call(
        _nms_body,
        out_shape=[
            jax.ShapeDtypeStruct((T, 1, B), jnp.float32),
            jax.ShapeDtypeStruct((T, 1, B), jnp.float32),
            jax.ShapeDtypeStruct((T, 4, B), jnp.float32),
            jax.ShapeDtypeStruct((1, B), jnp.int32),
        ],
        scratch_shapes=[
            pltpu.VMEM((B, C, NP), jnp.float32),
            pltpu.VMEM((T, B, C), jnp.float32),
            pltpu.VMEM((T, B, C), jnp.int32),
        ],
    )(conf_t, box_t)

    boxes_out = jnp.transpose(box_o, (2, 0, 1))
    conf_out = conf_o[:, 0, :].T
    cls_out = cls_o[:, 0, :].T
    num = num_o[0]
    return boxes_out, conf_out, cls_out, num
